# baseline (speedup 1.0000x reference)
import functools

import jax
import jax.numpy as jnp
from jax.experimental import pallas as pl

TOP_K = 2
N_EXPERTS = 8
D_MODEL = 1024
TOKENS_PER_BLOCK = 1024


def _router_kernel(x_ref, wt_ref, idx_ref, wgt_ref):
    x = x_ref[...]
    wt = wt_ref[...]
    logits = jnp.dot(x, wt, preferred_element_type=jnp.float32)

    m = jnp.max(logits, axis=1, keepdims=True)
    e = jnp.exp(logits - m)
    z = jnp.sum(e, axis=1, keepdims=True)
    s = e / z

    lane = jax.lax.broadcasted_iota(jnp.int32, s.shape, 1)
    v1 = jnp.max(s, axis=1, keepdims=True)
    i1 = jnp.min(jnp.where(s == v1, lane, N_EXPERTS), axis=1, keepdims=True)
    masked = jnp.where(lane == i1, -jnp.inf, s)
    v2 = jnp.max(masked, axis=1, keepdims=True)
    i2 = jnp.min(jnp.where(masked == v2, lane, N_EXPERTS), axis=1, keepdims=True)

    denom = v1 + v2 + 1e-20
    idx_ref[...] = jnp.concatenate([i1, i2], axis=1)
    wgt_ref[...] = jnp.concatenate([v1 / denom, v2 / denom], axis=1)


@jax.jit
def kernel(hidden_states, weight):
    h = hidden_states.shape[-1]
    x = hidden_states.reshape(-1, h).astype(jnp.float32)
    t = x.shape[0]
    wt = weight.astype(jnp.float32).T
    b = TOKENS_PER_BLOCK
    grid = (t // b,)
    idx, wgt = pl.pallas_call(
        _router_kernel,
        grid=grid,
        in_specs=[
            pl.BlockSpec((b, h), lambda i: (i, 0)),
            pl.BlockSpec((h, N_EXPERTS), lambda i: (0, 0)),
        ],
        out_specs=[
            pl.BlockSpec((b, TOP_K), lambda i: (i, 0)),
            pl.BlockSpec((b, TOP_K), lambda i: (i, 0)),
        ],
        out_shape=[
            jax.ShapeDtypeStruct((t, TOP_K), jnp.int32),
            jax.ShapeDtypeStruct((t, TOP_K), jnp.float32),
        ],
    )(x, wt)
    return (idx, wgt)


# speedup vs baseline: 1.0627x; 1.0627x over previous
import functools

import jax
import jax.numpy as jnp
from jax.experimental import pallas as pl

TOP_K = 2
N_EXPERTS = 8
D_MODEL = 1024
TOKENS_PER_BLOCK = 1024


def _router_kernel(x_ref, wt_ref, idx_ref, wgt_ref):
    x = x_ref[...]
    wt = wt_ref[...]
    logits = jnp.dot(x, wt, preferred_element_type=jnp.float32)

    lane = jax.lax.broadcasted_iota(jnp.int32, logits.shape, 1)
    v1 = jnp.max(logits, axis=1, keepdims=True)
    i1 = jnp.min(jnp.where(logits == v1, lane, N_EXPERTS), axis=1, keepdims=True)
    masked = jnp.where(lane == i1, -jnp.inf, logits)
    v2 = jnp.max(masked, axis=1, keepdims=True)
    i2 = jnp.min(jnp.where(masked == v2, lane, N_EXPERTS), axis=1, keepdims=True)

    d = jnp.exp(v2 - v1)
    w1 = 1.0 / (1.0 + d)
    idx_ref[...] = jnp.concatenate([i1, i2], axis=1)
    wgt_ref[...] = jnp.concatenate([w1, d * w1], axis=1)


@jax.jit
def kernel(hidden_states, weight):
    h = hidden_states.shape[-1]
    x = hidden_states.reshape(-1, h).astype(jnp.float32)
    t = x.shape[0]
    wt = weight.astype(jnp.float32).T
    b = TOKENS_PER_BLOCK
    grid = (t // b,)
    idx, wgt = pl.pallas_call(
        _router_kernel,
        grid=grid,
        in_specs=[
            pl.BlockSpec((b, h), lambda i: (i, 0)),
            pl.BlockSpec((h, N_EXPERTS), lambda i: (0, 0)),
        ],
        out_specs=[
            pl.BlockSpec((b, TOP_K), lambda i: (i, 0)),
            pl.BlockSpec((b, TOP_K), lambda i: (i, 0)),
        ],
        out_shape=[
            jax.ShapeDtypeStruct((t, TOP_K), jnp.int32),
            jax.ShapeDtypeStruct((t, TOP_K), jnp.float32),
        ],
    )(x, wt)
    return (idx, wgt)


# speedup vs baseline: 1.1783x; 1.1087x over previous
import functools

import jax
import jax.numpy as jnp
from jax.experimental import pallas as pl

TOP_K = 2
N_EXPERTS = 8
D_MODEL = 1024
TOKENS_PER_BLOCK = 2048


def _router_kernel(x_ref, wt_ref, idx_ref, wgt_ref):
    x = x_ref[...]
    wt = wt_ref[...]
    logits = jnp.dot(x, wt, preferred_element_type=jnp.float32)

    lane = jax.lax.broadcasted_iota(jnp.int32, logits.shape, 1)
    v1 = jnp.max(logits, axis=1, keepdims=True)
    i1 = jnp.min(jnp.where(logits == v1, lane, N_EXPERTS), axis=1, keepdims=True)
    masked = jnp.where(lane == i1, -jnp.inf, logits)
    v2 = jnp.max(masked, axis=1, keepdims=True)
    i2 = jnp.min(jnp.where(masked == v2, lane, N_EXPERTS), axis=1, keepdims=True)

    d = jnp.exp(v2 - v1)
    w1 = 1.0 / (1.0 + d)
    idx_ref[...] = jnp.concatenate([i1, i2], axis=1)
    wgt_ref[...] = jnp.concatenate([w1, d * w1], axis=1)


@jax.jit
def kernel(hidden_states, weight):
    h = hidden_states.shape[-1]
    x = hidden_states.reshape(-1, h).astype(jnp.float32)
    t = x.shape[0]
    wt = weight.astype(jnp.float32).T
    b = TOKENS_PER_BLOCK
    grid = (t // b,)
    idx, wgt = pl.pallas_call(
        _router_kernel,
        grid=grid,
        in_specs=[
            pl.BlockSpec((b, h), lambda i: (i, 0)),
            pl.BlockSpec((h, N_EXPERTS), lambda i: (0, 0)),
        ],
        out_specs=[
            pl.BlockSpec((b, TOP_K), lambda i: (i, 0)),
            pl.BlockSpec((b, TOP_K), lambda i: (i, 0)),
        ],
        out_shape=[
            jax.ShapeDtypeStruct((t, TOP_K), jnp.int32),
            jax.ShapeDtypeStruct((t, TOP_K), jnp.float32),
        ],
    )(x, wt)
    return (idx, wgt)


# speedup vs baseline: 1.2366x; 1.0495x over previous
import functools

import jax
import jax.numpy as jnp
from jax.experimental import pallas as pl

TOP_K = 2
N_EXPERTS = 8
D_MODEL = 1024
TOKENS_PER_BLOCK = 4096


def _router_kernel(x_ref, wt_ref, idx_ref, wgt_ref):
    x = x_ref[...]
    wt = wt_ref[...]
    logits = jnp.dot(x, wt, preferred_element_type=jnp.float32)

    lane = jax.lax.broadcasted_iota(jnp.int32, logits.shape, 1)
    v1 = jnp.max(logits, axis=1, keepdims=True)
    i1 = jnp.min(jnp.where(logits == v1, lane, N_EXPERTS), axis=1, keepdims=True)
    masked = jnp.where(lane == i1, -jnp.inf, logits)
    v2 = jnp.max(masked, axis=1, keepdims=True)
    i2 = jnp.min(jnp.where(masked == v2, lane, N_EXPERTS), axis=1, keepdims=True)

    d = jnp.exp(v2 - v1)
    w1 = 1.0 / (1.0 + d)
    idx_ref[...] = jnp.concatenate([i1, i2], axis=1)
    wgt_ref[...] = jnp.concatenate([w1, d * w1], axis=1)


@jax.jit
def kernel(hidden_states, weight):
    h = hidden_states.shape[-1]
    x = hidden_states.reshape(-1, h).astype(jnp.float32)
    t = x.shape[0]
    wt = weight.astype(jnp.float32).T
    b = TOKENS_PER_BLOCK
    grid = (t // b,)
    idx, wgt = pl.pallas_call(
        _router_kernel,
        grid=grid,
        in_specs=[
            pl.BlockSpec((b, h), lambda i: (i, 0)),
            pl.BlockSpec((h, N_EXPERTS), lambda i: (0, 0)),
        ],
        out_specs=[
            pl.BlockSpec((b, TOP_K), lambda i: (i, 0)),
            pl.BlockSpec((b, TOP_K), lambda i: (i, 0)),
        ],
        out_shape=[
            jax.ShapeDtypeStruct((t, TOP_K), jnp.int32),
            jax.ShapeDtypeStruct((t, TOP_K), jnp.float32),
        ],
    )(x, wt)
    return (idx, wgt)


# speedup vs baseline: 1.2625x; 1.0209x over previous
import jax
import jax.numpy as jnp
from jax.experimental import pallas as pl

TOP_K = 2
N_EXPERTS = 8
D_MODEL = 1024
NSTREAMS = 4
SUB_BLOCK = 1024
TOKENS_PER_BLOCK = NSTREAMS * SUB_BLOCK


def _router_kernel(*refs):
    x_refs = refs[:NSTREAMS]
    wt_ref, idx_ref, wgt_ref = refs[NSTREAMS:]
    wt = wt_ref[...]
    logits = jnp.concatenate(
        [jnp.dot(x_ref[...], wt, preferred_element_type=jnp.float32)
         for x_ref in x_refs], axis=0)

    lane = jax.lax.broadcasted_iota(jnp.int32, logits.shape, 1)
    v1 = jnp.max(logits, axis=1, keepdims=True)
    i1 = jnp.min(jnp.where(logits == v1, lane, N_EXPERTS), axis=1, keepdims=True)
    masked = jnp.where(lane == i1, -jnp.inf, logits)
    v2 = jnp.max(masked, axis=1, keepdims=True)
    i2 = jnp.min(jnp.where(masked == v2, lane, N_EXPERTS), axis=1, keepdims=True)

    d = jnp.exp(v2 - v1)
    w1 = 1.0 / (1.0 + d)
    idx_ref[...] = jnp.concatenate([i1, i2], axis=1)
    wgt_ref[...] = jnp.concatenate([w1, d * w1], axis=1)


@jax.jit
def kernel(hidden_states, weight):
    h = hidden_states.shape[-1]
    x = hidden_states.reshape(-1, h).astype(jnp.float32)
    t = x.shape[0]
    wt = weight.astype(jnp.float32).T
    b = TOKENS_PER_BLOCK
    grid = (t // b,)

    def make_spec(j):
        return pl.BlockSpec((SUB_BLOCK, h), lambda i, j=j: (i * NSTREAMS + j, 0))

    idx, wgt = pl.pallas_call(
        _router_kernel,
        grid=grid,
        in_specs=[make_spec(j) for j in range(NSTREAMS)] + [
            pl.BlockSpec((h, N_EXPERTS), lambda i: (0, 0)),
        ],
        out_specs=[
            pl.BlockSpec((b, TOP_K), lambda i: (i, 0)),
            pl.BlockSpec((b, TOP_K), lambda i: (i, 0)),
        ],
        out_shape=[
            jax.ShapeDtypeStruct((t, TOP_K), jnp.int32),
            jax.ShapeDtypeStruct((t, TOP_K), jnp.float32),
        ],
    )(*([x] * NSTREAMS), wt)
    return (idx, wgt)


# speedup vs baseline: 2.1801x; 1.7268x over previous
import jax
import jax.numpy as jnp
from jax.experimental import pallas as pl

TOP_K = 2
N_EXPERTS = 8
D_MODEL = 1024
NSTREAMS = 4
SUB_BLOCK = 1024
TOKENS_PER_BLOCK = NSTREAMS * SUB_BLOCK


def _router_kernel(*refs):
    x_refs = refs[:NSTREAMS]
    w_ref, idx_ref, wgt_ref = refs[NSTREAMS:]
    w = w_ref[...]
    dn = (((1,), (1,)), ((), ()))
    logits = jnp.concatenate(
        [jax.lax.dot_general(w, x_ref[...], dn,
                             preferred_element_type=jnp.float32)
         for x_ref in x_refs], axis=1)

    exp_row = jax.lax.broadcasted_iota(jnp.int32, logits.shape, 0)
    v1 = jnp.max(logits, axis=0, keepdims=True)
    i1 = jnp.min(jnp.where(logits == v1, exp_row, N_EXPERTS),
                 axis=0, keepdims=True)
    masked = jnp.where(exp_row == i1, -jnp.inf, logits)
    v2 = jnp.max(masked, axis=0, keepdims=True)
    i2 = jnp.min(jnp.where(masked == v2, exp_row, N_EXPERTS),
                 axis=0, keepdims=True)

    d = jnp.exp(v2 - v1)
    w1 = 1.0 / (1.0 + d)
    idx_ref[...] = jnp.concatenate([i1, i2], axis=0)
    wgt_ref[...] = jnp.concatenate([w1, d * w1], axis=0)


@jax.jit
def kernel(hidden_states, weight):
    h = hidden_states.shape[-1]
    x = hidden_states.reshape(-1, h).astype(jnp.float32)
    t = x.shape[0]
    w = weight.astype(jnp.float32)
    b = TOKENS_PER_BLOCK
    grid = (t // b,)

    def make_spec(j):
        return pl.BlockSpec((SUB_BLOCK, h), lambda i, j=j: (i * NSTREAMS + j, 0))

    idx_t, wgt_t = pl.pallas_call(
        _router_kernel,
        grid=grid,
        in_specs=[make_spec(j) for j in range(NSTREAMS)] + [
            pl.BlockSpec((N_EXPERTS, h), lambda i: (0, 0)),
        ],
        out_specs=[
            pl.BlockSpec((TOP_K, b), lambda i: (0, i)),
            pl.BlockSpec((TOP_K, b), lambda i: (0, i)),
        ],
        out_shape=[
            jax.ShapeDtypeStruct((TOP_K, t), jnp.int32),
            jax.ShapeDtypeStruct((TOP_K, t), jnp.float32),
        ],
    )(*([x] * NSTREAMS), w)
    return (idx_t.T, wgt_t.T)


# speedup vs baseline: 2.2788x; 1.0453x over previous
import jax
import jax.numpy as jnp
from jax.experimental import pallas as pl

TOP_K = 2
N_EXPERTS = 8
D_MODEL = 1024
NSTREAMS = 2
SUB_BLOCK = 1024
TOKENS_PER_BLOCK = NSTREAMS * SUB_BLOCK


def _router_kernel(*refs):
    x_refs = refs[:NSTREAMS]
    w_ref, idx_ref, wgt_ref = refs[NSTREAMS:]
    w = w_ref[...]
    dn = (((1,), (1,)), ((), ()))
    logits = jnp.concatenate(
        [jax.lax.dot_general(w, x_ref[...], dn,
                             preferred_element_type=jnp.float32)
         for x_ref in x_refs], axis=1)

    exp_row = jax.lax.broadcasted_iota(jnp.int32, logits.shape, 0)
    v1 = jnp.max(logits, axis=0, keepdims=True)
    i1 = jnp.min(jnp.where(logits == v1, exp_row, N_EXPERTS),
                 axis=0, keepdims=True)
    masked = jnp.where(exp_row == i1, -jnp.inf, logits)
    v2 = jnp.max(masked, axis=0, keepdims=True)
    i2 = jnp.min(jnp.where(masked == v2, exp_row, N_EXPERTS),
                 axis=0, keepdims=True)

    d = jnp.exp(v2 - v1)
    w1 = 1.0 / (1.0 + d)
    idx_ref[...] = jnp.concatenate([i1, i2], axis=0)
    wgt_ref[...] = jnp.concatenate([w1, d * w1], axis=0)


@jax.jit
def kernel(hidden_states, weight):
    h = hidden_states.shape[-1]
    x = hidden_states.reshape(-1, h).astype(jnp.float32)
    t = x.shape[0]
    w = weight.astype(jnp.float32)
    b = TOKENS_PER_BLOCK
    grid = (t // b,)

    def make_spec(j):
        return pl.BlockSpec((SUB_BLOCK, h), lambda i, j=j: (i * NSTREAMS + j, 0))

    idx_t, wgt_t = pl.pallas_call(
        _router_kernel,
        grid=grid,
        in_specs=[make_spec(j) for j in range(NSTREAMS)] + [
            pl.BlockSpec((N_EXPERTS, h), lambda i: (0, 0)),
        ],
        out_specs=[
            pl.BlockSpec((TOP_K, b), lambda i: (0, i)),
            pl.BlockSpec((TOP_K, b), lambda i: (0, i)),
        ],
        out_shape=[
            jax.ShapeDtypeStruct((TOP_K, t), jnp.int32),
            jax.ShapeDtypeStruct((TOP_K, t), jnp.float32),
        ],
    )(*([x] * NSTREAMS), w)
    return (idx_t.T, wgt_t.T)
